# baseline (speedup 1.0000x reference)
import functools
import jax
import jax.numpy as jnp
from jax import lax
from jax.experimental import pallas as pl
from jax.experimental.pallas import tpu as pltpu
from jax.experimental.pallas import tpu_sc as plsc

DIM_POS = 64
DIM_TOK = 128
DIM_OUT = DIM_POS + DIM_TOK

_NC = 2
_NS = 16
_NW = _NC * _NS
_NSLOT = 4
_LANES = 16


def _make_kernel(batch, seq):
    assert batch % _NW == 0
    bpw = batch // _NW
    n_iter = bpw + _NSLOT
    mesh = plsc.VectorSubcoreMesh(core_axis_name="c", subcore_axis_name="s")

    @functools.partial(
        pl.kernel,
        out_type=jax.ShapeDtypeStruct((batch, seq, DIM_OUT), jnp.float32),
        mesh=mesh,
        scratch_types=[
            pltpu.VMEM((bpw, seq), jnp.int32),
            pltpu.VMEM((bpw, seq), jnp.int32),
            [pltpu.VMEM((seq, DIM_OUT), jnp.float32) for _ in range(_NSLOT)],
            [pltpu.VMEM((seq, DIM_TOK), jnp.float32) for _ in range(_NSLOT)],
            [pltpu.SemaphoreType.DMA for _ in range(_NSLOT)],
            [pltpu.SemaphoreType.DMA for _ in range(_NSLOT)],
        ],
    )
    def embed(tok_hbm, pos_hbm, wt_hbm, wp_hbm, out_hbm,
              tok_idx, pos_idx, comb_bufs, tok_bufs, gsems, osems):
        wid = lax.axis_index("s") * _NC + lax.axis_index("c")
        b0 = wid * bpw

        pltpu.sync_copy(tok_hbm.at[pl.ds(b0, bpw)], tok_idx)
        pltpu.sync_copy(pos_hbm.at[pl.ds(b0, bpw)], pos_idx)

        def issue_gather(g, s):
            pltpu.async_copy(wp_hbm.at[pos_idx.at[g]],
                             comb_bufs[s].at[:, pl.ds(0, DIM_TOK)], gsems[s])
            pltpu.async_copy(wt_hbm.at[tok_idx.at[g]], tok_bufs[s], gsems[s])

        def drain_gather(g, s):
            pltpu.make_async_copy(wp_hbm.at[pos_idx.at[g]],
                                  comb_bufs[s].at[:, pl.ds(0, DIM_TOK)],
                                  gsems[s]).wait()
            pltpu.make_async_copy(wt_hbm.at[tok_idx.at[g]], tok_bufs[s],
                                  gsems[s]).wait()

        def fixup(s):
            comb = comb_bufs[s]
            tokb = tok_bufs[s]

            def row(r, carry):
                for c in range(DIM_TOK // _LANES):
                    comb[r, pl.ds(DIM_POS + c * _LANES, _LANES)] = (
                        tokb[r, pl.ds(c * _LANES, _LANES)])
                return carry

            lax.fori_loop(0, seq, row, 0)

        def issue_out(g, s):
            pltpu.async_copy(comb_bufs[s], out_hbm.at[b0 + g], osems[s])

        def drain_out(s):
            pltpu.make_async_copy(comb_bufs[s], out_hbm.at[0], osems[s]).wait()

        def body(j, carry):
            for k in range(_NSLOT):
                i = j * _NSLOT + k

                @pl.when((i >= _NSLOT) & (i < bpw + _NSLOT))
                def _(i=i, k=k):
                    drain_out(k)

                @pl.when(i < bpw)
                def _(i=i, k=k):
                    issue_gather(i, k)

                @pl.when((i >= 2) & (i < bpw + 2))
                def _(i=i, k=k):
                    s = (k + _NSLOT - 2) % _NSLOT
                    drain_gather(i - 2, s)
                    fixup(s)
                    issue_out(i - 2, s)

            return carry

        lax.fori_loop(0, (n_iter + _NSLOT - 1) // _NSLOT, body, 0)

    return embed


_POS_REP = 8


def kernel(tokens, pos, W_tokens, W_pos):
    batch, seq = tokens.shape
    vocab_pos = W_pos.shape[0]
    wp_pad = jnp.pad(W_pos, ((0, 0), (0, DIM_TOK - DIM_POS)))
    wp_rep = jnp.tile(wp_pad, (_POS_REP, 1))
    bpw = batch // _NW
    rep = ((jnp.arange(batch, dtype=jnp.int32) // bpw) % _POS_REP) * vocab_pos
    pos_adj = pos.astype(jnp.int32) + rep[:, None]
    return _make_kernel(batch, seq)(
        tokens.astype(jnp.int32), pos_adj, W_tokens, wp_rep)
